# baseline (speedup 1.0000x reference)
import jax
import jax.numpy as jnp
from jax import lax
from jax.experimental import pallas as pl
from jax.experimental.pallas import tpu as pltpu
from jax.experimental.pallas import tpu_sc as plsc

_VOCAB = 1000000
_D = 32
_B = 16384
_L = 50
_T = _B * _L
_H1 = 256
_H2 = 128
_OUT = 4

_NC = 2
_NS = 16
_NW = _NC * _NS

_CW = 512
_NBLK = -(-_VOCAB // _CW)
_OROWS = _NBLK * 128


def _transp_body(x_ref, o_ref):
    y = x_ref[...].T
    for a in range(4):
        o_ref[:, pl.ds(32 * a, 32)] = y[128 * a:128 * (a + 1), :]


_transp = pl.pallas_call(
    _transp_body,
    grid=(_NBLK,),
    in_specs=[pl.BlockSpec((_D, _CW), lambda i: (0, i))],
    out_specs=pl.BlockSpec((128, 128), lambda i: (i, 0)),
    out_shape=jax.ShapeDtypeStruct((_OROWS, 128), jnp.float32),
)


_BAGS_W = _B // _NW
_CB = 32
_NCHUNK = _BAGS_W // _CB
_CT = _CB * _L
_GW = 80
_G = _CT // _GW


def _embag_body(text_hbm, table_hbm, out_hbm,
                idx_v0, idx_v1, rows_v0, rows_v1, out_v,
                isem0, isem1, gsem0, gsem1):
    wid = lax.axis_index("s") * _NC + lax.axis_index("c")
    bag0 = wid * _BAGS_W
    tok0 = bag0 * _L
    idx_bufs = (idx_v0, idx_v1)
    rows_bufs = (rows_v0, rows_v1)
    isems = (isem0, isem1)
    gsems = (gsem0, gsem1)

    def fetch(t, s):
        idx_v, rows_v = idx_bufs[s], rows_bufs[s]
        pltpu.async_copy(
            text_hbm.at[pl.ds(tok0 + t * _CT, _CT)], idx_v, isems[s]
        ).wait()
        for j in range(_G):
            pltpu.async_copy(
                table_hbm.at[idx_v.at[pl.ds(j * _GW, _GW)]],
                rows_v.at[pl.ds(j * _GW, _GW)],
                gsems[s],
            )

    def drain(s):
        rows_v = rows_bufs[s]
        for j in range(_G):
            pltpu.make_async_copy(
                table_hbm.at[idx_bufs[s].at[pl.ds(j * _GW, _GW)]],
                rows_v.at[pl.ds(j * _GW, _GW)],
                gsems[s],
            ).wait()

    def reduce_chunk(t, s):
        rows_v = rows_bufs[s]

        def bag_body(i, c):
            r0 = i * _L
            acc0 = rows_v[r0, pl.ds(0, 16)]
            acc1 = rows_v[r0, pl.ds(16, 16)]
            for r in range(1, _L):
                acc0 = acc0 + rows_v[r0 + r, pl.ds(0, 16)]
                acc1 = acc1 + rows_v[r0 + r, pl.ds(16, 16)]
            out_v[i, pl.ds(0, 16)] = acc0 * (1.0 / _L)
            out_v[i, pl.ds(16, 16)] = acc1 * (1.0 / _L)
            return c

        lax.fori_loop(0, _CB, bag_body, 0)
        pltpu.sync_copy(out_v, out_hbm.at[pl.ds(bag0 + t * _CB, _CB)])

    fetch(0, 0)

    @pl.loop(0, _NCHUNK, step=2)
    def _chunk_pair(t0):
        for b in range(2):
            t = t0 + b

            @pl.when(t + 1 < _NCHUNK)
            def _prefetch():
                fetch(t + 1, 1 - b)

            drain(b)
            reduce_chunk(t, b)


_embag = pl.kernel(
    _embag_body,
    out_type=jax.ShapeDtypeStruct((_B, _D), jnp.float32),
    mesh=plsc.VectorSubcoreMesh(
        core_axis_name="c", subcore_axis_name="s",
        num_cores=_NC, num_subcores=_NS,
    ),
    scratch_types=[
        pltpu.VMEM((_CT,), jnp.int32),
        pltpu.VMEM((_CT,), jnp.int32),
        pltpu.VMEM((_CT, _D), jnp.float32),
        pltpu.VMEM((_CT, _D), jnp.float32),
        pltpu.VMEM((_CB, _D), jnp.float32),
        pltpu.SemaphoreType.DMA,
        pltpu.SemaphoreType.DMA,
        pltpu.SemaphoreType.DMA,
        pltpu.SemaphoreType.DMA,
    ],
    compiler_params=pltpu.CompilerParams(use_tc_tiling_on_sc=False),
)


_BLK = 1024


def _mlp_body(x_ref, w1_ref, b1_ref, w2_ref, b2_ref, w3_ref, b3_ref, o_ref):
    h = lax.dot_general(
        x_ref[...], w1_ref[...], (((1,), (1,)), ((), ())),
        preferred_element_type=jnp.float32,
    )
    h = jnp.maximum(h + b1_ref[...], 0.0)
    h = lax.dot_general(
        h, w2_ref[...], (((1,), (1,)), ((), ())),
        preferred_element_type=jnp.float32,
    )
    h = jnp.maximum(h + b2_ref[...], 0.0)
    o_ref[...] = lax.dot_general(
        h, w3_ref[...], (((1,), (1,)), ((), ())),
        preferred_element_type=jnp.float32,
    ) + b3_ref[...]


_mlp = pl.pallas_call(
    _mlp_body,
    grid=(_B // _BLK,),
    in_specs=[
        pl.BlockSpec((_BLK, _D), lambda i: (i, 0)),
        pl.BlockSpec((_H1, _D), lambda i: (0, 0)),
        pl.BlockSpec((1, _H1), lambda i: (0, 0)),
        pl.BlockSpec((_H2, _H1), lambda i: (0, 0)),
        pl.BlockSpec((1, _H2), lambda i: (0, 0)),
        pl.BlockSpec((_OUT, _H2), lambda i: (0, 0)),
        pl.BlockSpec((1, _OUT), lambda i: (0, 0)),
    ],
    out_specs=pl.BlockSpec((_BLK, _OUT), lambda i: (i, 0)),
    out_shape=jax.ShapeDtypeStruct((_B, _OUT), jnp.float32),
)


@jax.jit
def _run(text, table, W1, b1, W2, b2, W3, b3):
    tlin = _transp(table.T)
    gtext = (text // 512) * 512 + (text % 128) * 4 + (text % 512) // 128
    emb = _embag(gtext, tlin.reshape(_OROWS * 4, _D))
    return _mlp(emb, W1, b1[None, :], W2, b2[None, :], W3, b3[None, :])


def kernel(text, offsets, table, W1, b1, W2, b2, W3, b3):
    return _run(text, table, W1, b1, W2, b2, W3, b3)


# speedup vs baseline: 2.1246x; 2.1246x over previous
import jax
import jax.numpy as jnp
from jax import lax
from jax.experimental import pallas as pl
from jax.experimental.pallas import tpu as pltpu
from jax.experimental.pallas import tpu_sc as plsc

_VOCAB = 1000000
_D = 32
_B = 16384
_L = 50
_T = _B * _L
_H1 = 256
_H2 = 128
_OUT = 4

_NC = 2
_NS = 16
_NW = _NC * _NS
_BAGS_W = _B // _NW
_CB = 32
_NCHUNK = _BAGS_W // _CB
_CT = _CB * _L
_GW = 80
_G = _CT // _GW


def _embag_body(text_hbm, table_hbm, out_hbm,
                idx_v0, idx_v1, rows_v0, rows_v1, out_v,
                isem0, isem1, gsem0, gsem1):
    wid = lax.axis_index("s") * _NC + lax.axis_index("c")
    bag0 = wid * _BAGS_W
    tok0 = bag0 * _L
    idx_bufs = (idx_v0, idx_v1)
    rows_bufs = (rows_v0, rows_v1)
    isems = (isem0, isem1)
    gsems = (gsem0, gsem1)

    def fetch(t, s):
        idx_v, rows_v = idx_bufs[s], rows_bufs[s]
        pltpu.async_copy(
            text_hbm.at[pl.ds(tok0 + t * _CT, _CT)], idx_v, isems[s]
        ).wait()
        for j in range(_G):
            pltpu.async_copy(
                table_hbm.at[idx_v.at[pl.ds(j * _GW, _GW)]],
                rows_v.at[pl.ds(j * _GW, _GW)],
                gsems[s],
            )

    def drain(s):
        rows_v = rows_bufs[s]
        for j in range(_G):
            pltpu.make_async_copy(
                table_hbm.at[idx_bufs[s].at[pl.ds(j * _GW, _GW)]],
                rows_v.at[pl.ds(j * _GW, _GW)],
                gsems[s],
            ).wait()

    def reduce_chunk(t, s):
        rows_v = rows_bufs[s]

        def bag_body(i, c):
            r0 = i * _L
            acc0 = rows_v[r0, pl.ds(0, 16)]
            acc1 = rows_v[r0, pl.ds(16, 16)]
            for r in range(1, _L):
                acc0 = acc0 + rows_v[r0 + r, pl.ds(0, 16)]
                acc1 = acc1 + rows_v[r0 + r, pl.ds(16, 16)]
            out_v[i, pl.ds(0, 16)] = acc0 * (1.0 / _L)
            out_v[i, pl.ds(16, 16)] = acc1 * (1.0 / _L)
            return c

        lax.fori_loop(0, _CB, bag_body, 0)
        pltpu.sync_copy(out_v, out_hbm.at[pl.ds(bag0 + t * _CB, _CB)])

    fetch(0, 0)

    @pl.loop(0, _NCHUNK, step=2)
    def _chunk_pair(t0):
        for b in range(2):
            t = t0 + b

            @pl.when(t + 1 < _NCHUNK)
            def _prefetch():
                fetch(t + 1, 1 - b)

            drain(b)
            reduce_chunk(t, b)


_embag = pl.kernel(
    _embag_body,
    out_type=jax.ShapeDtypeStruct((_B, _D), jnp.float32),
    mesh=plsc.VectorSubcoreMesh(
        core_axis_name="c", subcore_axis_name="s",
        num_cores=_NC, num_subcores=_NS,
    ),
    scratch_types=[
        pltpu.VMEM((_CT,), jnp.int32),
        pltpu.VMEM((_CT,), jnp.int32),
        pltpu.VMEM((_CT, _D), jnp.float32),
        pltpu.VMEM((_CT, _D), jnp.float32),
        pltpu.VMEM((_CB, _D), jnp.float32),
        pltpu.SemaphoreType.DMA,
        pltpu.SemaphoreType.DMA,
        pltpu.SemaphoreType.DMA,
        pltpu.SemaphoreType.DMA,
    ],
    compiler_params=pltpu.CompilerParams(use_tc_tiling_on_sc=False),
)


_BLK = 1024


def _mlp_body(x_ref, w1_ref, b1_ref, w2_ref, b2_ref, w3_ref, b3_ref, o_ref):
    h = lax.dot_general(
        x_ref[...], w1_ref[...], (((1,), (1,)), ((), ())),
        preferred_element_type=jnp.float32,
    )
    h = jnp.maximum(h + b1_ref[...], 0.0)
    h = lax.dot_general(
        h, w2_ref[...], (((1,), (1,)), ((), ())),
        preferred_element_type=jnp.float32,
    )
    h = jnp.maximum(h + b2_ref[...], 0.0)
    o_ref[...] = lax.dot_general(
        h, w3_ref[...], (((1,), (1,)), ((), ())),
        preferred_element_type=jnp.float32,
    ) + b3_ref[...]


_mlp = pl.pallas_call(
    _mlp_body,
    grid=(_B // _BLK,),
    in_specs=[
        pl.BlockSpec((_BLK, _D), lambda i: (i, 0)),
        pl.BlockSpec((_H1, _D), lambda i: (0, 0)),
        pl.BlockSpec((1, _H1), lambda i: (0, 0)),
        pl.BlockSpec((_H2, _H1), lambda i: (0, 0)),
        pl.BlockSpec((1, _H2), lambda i: (0, 0)),
        pl.BlockSpec((_OUT, _H2), lambda i: (0, 0)),
        pl.BlockSpec((1, _OUT), lambda i: (0, 0)),
    ],
    out_specs=pl.BlockSpec((_BLK, _OUT), lambda i: (i, 0)),
    out_shape=jax.ShapeDtypeStruct((_B, _OUT), jnp.float32),
)


@jax.jit
def _run(text, table, W1, b1, W2, b2, W3, b3):
    emb = _embag(text, table)
    return _mlp(emb, W1, b1[None, :], W2, b2[None, :], W3, b3[None, :])


def kernel(text, offsets, table, W1, b1, W2, b2, W3, b3):
    return _run(text, table, W1, b1, W2, b2, W3, b3)


# speedup vs baseline: 2.1548x; 1.0142x over previous
import jax
import jax.numpy as jnp
from jax import lax
from jax.experimental import pallas as pl
from jax.experimental.pallas import tpu as pltpu
from jax.experimental.pallas import tpu_sc as plsc

_VOCAB = 1000000
_D = 32
_B = 16384
_L = 50
_T = _B * _L
_H1 = 256
_H2 = 128
_OUT = 4

_NC = 2
_NS = 16
_NW = _NC * _NS
_BAGS_W = _B // _NW
_CB = 32
_NCHUNK = _BAGS_W // _CB
_CT = _CB * _L
_GW = 80
_G = _CT // _GW


def _embag_body(text_hbm, table_hbm, out_hbm,
                idx_v0, idx_v1, rows_v0, rows_v1, out_v,
                isem0, isem1, gsem0, gsem1):
    wid = lax.axis_index("s") * _NC + lax.axis_index("c")
    bag0 = wid * _BAGS_W
    tok0 = bag0 * _L
    idx_bufs = (idx_v0, idx_v1)
    rows_bufs = (rows_v0, rows_v1)
    isems = (isem0, isem1)
    gsems = (gsem0, gsem1)

    def fetch(t, s):
        idx_v, rows_v = idx_bufs[s], rows_bufs[s]
        pltpu.async_copy(
            text_hbm.at[pl.ds(tok0 + t * _CT, _CT)], idx_v, isems[s]
        ).wait()
        for j in range(_G):
            pltpu.async_copy(
                table_hbm.at[idx_v.at[pl.ds(j * _GW, _GW)]],
                rows_v.at[pl.ds(j * _GW, _GW)],
                gsems[s],
            )

    def drain(s):
        rows_v = rows_bufs[s]
        for j in range(_G):
            pltpu.make_async_copy(
                table_hbm.at[idx_bufs[s].at[pl.ds(j * _GW, _GW)]],
                rows_v.at[pl.ds(j * _GW, _GW)],
                gsems[s],
            ).wait()

    def reduce_chunk(t, s):
        rows_v = rows_bufs[s]

        def bag_body(i, c):
            r0 = i * _L
            acc0 = rows_v[r0, pl.ds(0, 16)]
            acc1 = rows_v[r0, pl.ds(16, 16)]
            for r in range(1, _L):
                acc0 = acc0 + rows_v[r0 + r, pl.ds(0, 16)]
                acc1 = acc1 + rows_v[r0 + r, pl.ds(16, 16)]
            out_v[i, pl.ds(0, 16)] = acc0 * (1.0 / _L)
            out_v[i, pl.ds(16, 16)] = acc1 * (1.0 / _L)
            return c

        lax.fori_loop(0, _CB, bag_body, 0)
        pltpu.sync_copy(out_v, out_hbm.at[pl.ds(bag0 + t * _CB, _CB)])

    fetch(0, 0)

    @pl.loop(0, _NCHUNK, step=2)
    def _chunk_pair(t0):
        for b in range(2):
            t = t0 + b

            @pl.when(t + 1 < _NCHUNK)
            def _prefetch():
                fetch(t + 1, 1 - b)

            drain(b)
            reduce_chunk(t, b)


_embag = pl.kernel(
    _embag_body,
    out_type=jax.ShapeDtypeStruct((_B, _D), jnp.float32),
    mesh=plsc.VectorSubcoreMesh(
        core_axis_name="c", subcore_axis_name="s",
        num_cores=_NC, num_subcores=_NS,
    ),
    scratch_types=[
        pltpu.VMEM((_CT,), jnp.int32),
        pltpu.VMEM((_CT,), jnp.int32),
        pltpu.VMEM((_CT, _D), jnp.float32),
        pltpu.VMEM((_CT, _D), jnp.float32),
        pltpu.VMEM((_CB, _D), jnp.float32),
        pltpu.SemaphoreType.DMA,
        pltpu.SemaphoreType.DMA,
        pltpu.SemaphoreType.DMA,
        pltpu.SemaphoreType.DMA,
    ],
    compiler_params=pltpu.CompilerParams(use_tc_tiling_on_sc=False),
)


_BLK = 4096


def _mlp_body(x_ref, w1_ref, b1_ref, w2_ref, b2_ref, w3_ref, b3_ref, o_ref):
    h = lax.dot_general(
        x_ref[...], w1_ref[...], (((1,), (1,)), ((), ())),
        preferred_element_type=jnp.float32,
    )
    h = jnp.maximum(h + b1_ref[...], 0.0)
    h = lax.dot_general(
        h, w2_ref[...], (((1,), (1,)), ((), ())),
        preferred_element_type=jnp.float32,
    )
    h = jnp.maximum(h + b2_ref[...], 0.0)
    o_ref[...] = lax.dot_general(
        h, w3_ref[...], (((1,), (1,)), ((), ())),
        preferred_element_type=jnp.float32,
    ) + b3_ref[...]


_mlp = pl.pallas_call(
    _mlp_body,
    grid=(_B // _BLK,),
    in_specs=[
        pl.BlockSpec((_BLK, _D), lambda i: (i, 0)),
        pl.BlockSpec((_H1, _D), lambda i: (0, 0)),
        pl.BlockSpec((1, _H1), lambda i: (0, 0)),
        pl.BlockSpec((_H2, _H1), lambda i: (0, 0)),
        pl.BlockSpec((1, _H2), lambda i: (0, 0)),
        pl.BlockSpec((_OUT, _H2), lambda i: (0, 0)),
        pl.BlockSpec((1, _OUT), lambda i: (0, 0)),
    ],
    out_specs=pl.BlockSpec((_BLK, _OUT), lambda i: (i, 0)),
    out_shape=jax.ShapeDtypeStruct((_B, _OUT), jnp.float32),
)


@jax.jit
def _run(text, table, W1, b1, W2, b2, W3, b3):
    emb = _embag(text, table)
    return _mlp(emb, W1, b1[None, :], W2, b2[None, :], W3, b3[None, :])


def kernel(text, offsets, table, W1, b1, W2, b2, W3, b3):
    return _run(text, table, W1, b1, W2, b2, W3, b3)
